# Initial kernel scaffold; baseline (speedup 1.0000x reference)
import jax
import jax.numpy as jnp
from jax.experimental import pallas as pl


def kernel(x, edge_index_rel0, edge_index_rel1, W0, b0, W1, b1):
    raise NotImplementedError("write your pallas kernel here")


# speedup vs baseline: 4.4920x; 4.4920x over previous
import functools

import jax
import jax.numpy as jnp
from jax import lax
from jax.experimental import pallas as pl
from jax.experimental.pallas import tpu as pltpu
from jax.experimental.pallas import tpu_sc as plsc

_N = 10000
_E = 160000
_D = 128

_NC = 2
_NS = 16

_EPT = _E // _NS
_CH = 80
_NG = _CH // 16
_NCHUNK = _EPT // _CH
_NPAD = 10240
_NBLK = _NPAD // 128
_ZROWS = _NPAD // _NS

_BLK = 2048
_GRID = _NPAD // _BLK
_BB = _BLK // 128

_mesh = plsc.VectorSubcoreMesh(
    core_axis_name="c", subcore_axis_name="s", num_cores=_NC, num_subcores=_NS
)


def _deg_body(srcs, dsts, zeros_n, deg_out, s_idx, d_idx, h_src, h_dst):
    c = lax.axis_index("c")
    s = lax.axis_index("s")
    pltpu.sync_copy(zeros_n, h_src)
    pltpu.sync_copy(zeros_n, h_dst)
    ones16 = jnp.ones((16,), jnp.float32)

    def body(j, carry):
        eb = c * _E + s * _EPT + j * _CH
        pltpu.sync_copy(srcs.at[pl.ds(eb, _CH)], s_idx)
        pltpu.sync_copy(dsts.at[pl.ds(eb, _CH)], d_idx)
        for g in range(_NG):
            plsc.addupdate_scatter(h_src, [s_idx[pl.ds(g * 16, 16)]], ones16)
            plsc.addupdate_scatter(h_dst, [d_idx[pl.ds(g * 16, 16)]], ones16)
        return carry

    lax.fori_loop(0, _NCHUNK, body, 0)
    base = ((c * 2 + 0) * _NS + s) * _NPAD
    pltpu.sync_copy(h_src, deg_out.at[pl.ds(base, _NPAD)])
    base1 = ((c * 2 + 1) * _NS + s) * _NPAD
    pltpu.sync_copy(h_dst, deg_out.at[pl.ds(base1, _NPAD)])


def _make_deg_kernel(interpret=False):
    return functools.partial(
        pl.kernel,
        out_type=jax.ShapeDtypeStruct((2 * 2 * _NS * _NPAD,), jnp.float32),
        mesh=_mesh,
        scratch_types=[
            pltpu.VMEM((_CH,), jnp.int32),
            pltpu.VMEM((_CH,), jnp.int32),
            pltpu.VMEM((_NPAD,), jnp.float32),
            pltpu.VMEM((_NPAD,), jnp.float32),
        ],
        compiler_params=pltpu.CompilerParams(needs_layout_passes=False),
        interpret=interpret,
    )(_deg_body)


_deg_kernel = _make_deg_kernel()


def _agg_body(y2, src_g, dsts, zbig, agg_out, s_idx, d_idx, rows_v, zbuf, agg_sh):
    c = lax.axis_index("c")
    s = lax.axis_index("s")
    pltpu.sync_copy(zbig, zbuf)
    for k in range(_ZROWS // 128):
        pltpu.sync_copy(zbuf, agg_sh.at[pl.ds(s * _ZROWS + k * 128, 128)])
    plsc.subcore_barrier()

    def body(j, carry):
        eb = c * _E + s * _EPT + j * _CH
        pltpu.sync_copy(src_g.at[pl.ds(eb, _CH)], s_idx)
        pltpu.sync_copy(dsts.at[pl.ds(eb, _CH)], d_idx)
        pltpu.sync_copy(y2.at[s_idx], rows_v)
        pltpu.sync_copy(rows_v, agg_sh.at[d_idx], add=True)
        return carry

    lax.fori_loop(0, _NCHUNK, body, 0)
    plsc.subcore_barrier()
    pltpu.sync_copy(
        agg_sh.at[pl.ds(s * _ZROWS, _ZROWS)],
        agg_out.at[c, pl.ds(s * _ZROWS, _ZROWS)],
    )


def _make_agg_kernel(interpret=False):
    return functools.partial(
        pl.kernel,
        out_type=jax.ShapeDtypeStruct((2, _NPAD, _D), jnp.float32),
        mesh=_mesh,
        scratch_types=[
            pltpu.VMEM((_CH,), jnp.int32),
            pltpu.VMEM((_CH,), jnp.int32),
            pltpu.VMEM((_CH, _D), jnp.float32),
            pltpu.VMEM((128, _D), jnp.float32),
            pltpu.VMEM_SHARED((_NPAD, _D), jnp.float32),
        ],
        interpret=interpret,
    )(_agg_body)


_agg_kernel = _make_agg_kernel()


def _scale_body(x_ref, deg_ref, y_ref):
    x3 = x_ref[...].reshape(_BB, 128, _D)
    dsum = jnp.sum(deg_ref[...], axis=2)
    cs = lax.rsqrt(jnp.maximum(dsum[:, 0], 1.0))
    y_ref[0] = (x3 * cs[0][:, :, None]).reshape(_BLK, _D)
    y_ref[1] = (x3 * cs[1][:, :, None]).reshape(_BLK, _D)


def _scale(x, deg_t):
    return pl.pallas_call(
        _scale_body,
        grid=(_GRID,),
        in_specs=[
            pl.BlockSpec((_BLK, _D), lambda i: (i, 0)),
            pl.BlockSpec((2, 2, _NS, _BB, 128), lambda i: (0, 0, 0, i, 0)),
        ],
        out_specs=pl.BlockSpec((2, _BLK, _D), lambda i: (0, i, 0)),
        out_shape=jax.ShapeDtypeStruct((2, _N, _D), jnp.float32),
    )(x, deg_t)


def _combine_body(agg_ref, deg_ref, w0_ref, w1_ref, b0_ref, b1_ref, o_ref):
    dsum = jnp.sum(deg_ref[...], axis=2)
    ci = lax.rsqrt(jnp.maximum(dsum[:, 1], 1.0))
    a = agg_ref[...]
    z0 = (a[0].reshape(_BB, 128, _D) * ci[0][:, :, None]).reshape(_BLK, _D)
    z1 = (a[1].reshape(_BB, 128, _D) * ci[1][:, :, None]).reshape(_BLK, _D)
    o_ref[...] = 0.5 * (
        jnp.dot(z0, w0_ref[...], preferred_element_type=jnp.float32)
        + jnp.dot(z1, w1_ref[...], preferred_element_type=jnp.float32)
        + (b0_ref[...] + b1_ref[...])
    )


def _combine(agg, deg_t, W0, W1, b0, b1):
    return pl.pallas_call(
        _combine_body,
        grid=(_GRID,),
        in_specs=[
            pl.BlockSpec((2, _BLK, _D), lambda i: (0, i, 0)),
            pl.BlockSpec((2, 2, _NS, _BB, 128), lambda i: (0, 0, 0, i, 0)),
            pl.BlockSpec((_D, _D), lambda i: (0, 0)),
            pl.BlockSpec((_D, _D), lambda i: (0, 0)),
            pl.BlockSpec((1, _D), lambda i: (0, 0)),
            pl.BlockSpec((1, _D), lambda i: (0, 0)),
        ],
        out_specs=pl.BlockSpec((_BLK, _D), lambda i: (i, 0)),
        out_shape=jax.ShapeDtypeStruct((_N, _D), jnp.float32),
    )(agg, deg_t, W0, W1, b0, b1)


def kernel(x, edge_index_rel0, edge_index_rel1, W0, b0, W1, b1):
    srcs = jnp.concatenate([edge_index_rel0[0], edge_index_rel1[0]])
    dsts = jnp.concatenate([edge_index_rel0[1], edge_index_rel1[1]])
    src_g = jnp.concatenate([edge_index_rel0[0], edge_index_rel1[0] + _N])
    zeros_n = jnp.zeros((_NPAD,), jnp.float32)
    zbig = jnp.zeros((128, _D), jnp.float32)

    deg_t = _deg_kernel(srcs, dsts, zeros_n).reshape(2, 2, _NS, _NBLK, 128)
    y = _scale(x, deg_t)
    agg = _agg_kernel(y.reshape(2 * _N, _D), src_g, dsts, zbig)
    return _combine(agg, deg_t, W0, W1, b0[None, :], b1[None, :])


# speedup vs baseline: 10.7682x; 2.3972x over previous
import functools

import jax
import jax.numpy as jnp
from jax import lax
from jax.experimental import pallas as pl
from jax.experimental.pallas import tpu as pltpu
from jax.experimental.pallas import tpu_sc as plsc

_N = 10000
_E = 160000
_D = 128

_NC = 2
_NS = 16

_EPT = _E // _NS
_CH = 80
_NG = _CH // 16
_NCHUNK = _EPT // _CH
_NPAD = 10240
_NBLK = _NPAD // 128
_ZROWS = _NPAD // _NS

_BLK = 2048
_GRID = _NPAD // _BLK
_BB = _BLK // 128

_mesh = plsc.VectorSubcoreMesh(
    core_axis_name="c", subcore_axis_name="s", num_cores=_NC, num_subcores=_NS
)


def _deg_body(srcs4, dsts4, zeros_n, deg_out, sidx2, didx2, h_src, h_dst):
    c = lax.axis_index("c")
    s = lax.axis_index("s")
    pltpu.sync_copy(srcs4.at[c, s], sidx2)
    pltpu.sync_copy(dsts4.at[c, s], didx2)
    pltpu.sync_copy(zeros_n, h_src)
    pltpu.sync_copy(zeros_n, h_dst)
    ones16 = jnp.ones((16,), jnp.float32)

    def body(j, carry):
        for g in range(_NG):
            plsc.addupdate_scatter(h_src, [sidx2[j, pl.ds(g * 16, 16)]], ones16)
            plsc.addupdate_scatter(h_dst, [didx2[j, pl.ds(g * 16, 16)]], ones16)
        return carry

    lax.fori_loop(0, _NCHUNK, body, 0)
    base = ((c * 2 + 0) * _NS + s) * _NPAD
    pltpu.sync_copy(h_src, deg_out.at[pl.ds(base, _NPAD)])
    base1 = ((c * 2 + 1) * _NS + s) * _NPAD
    pltpu.sync_copy(h_dst, deg_out.at[pl.ds(base1, _NPAD)])


def _make_deg_kernel(interpret=False):
    return functools.partial(
        pl.kernel,
        out_type=jax.ShapeDtypeStruct((2 * 2 * _NS * _NPAD,), jnp.float32),
        mesh=_mesh,
        scratch_types=[
            pltpu.VMEM((_NCHUNK, _CH), jnp.int32),
            pltpu.VMEM((_NCHUNK, _CH), jnp.int32),
            pltpu.VMEM((_NPAD,), jnp.float32),
            pltpu.VMEM((_NPAD,), jnp.float32),
        ],
        compiler_params=pltpu.CompilerParams(needs_layout_passes=False),
        interpret=interpret,
    )(_deg_body)


_deg_kernel = _make_deg_kernel()


def _agg_body(y2, src_g4, dsts, zbig, agg_out,
              sidx2, di0, di1, buf0, buf1, agg_sh, sem0, sem1, sd0, sd1):
    c = lax.axis_index("c")
    s = lax.axis_index("s")
    pltpu.sync_copy(src_g4.at[c, s], sidx2)
    for k in range(_ZROWS // 128):
        pltpu.sync_copy(zbig, agg_sh.at[pl.ds(s * _ZROWS + k * 128, 128)])
    plsc.subcore_barrier()
    eb = c * _E + s * _EPT

    pltpu.async_copy(dsts.at[pl.ds(eb, _CH)], di0, sd0)
    pltpu.async_copy(dsts.at[pl.ds(eb + _CH, _CH)], di1, sd1)
    pltpu.async_copy(y2.at[sidx2.at[0]], buf0, sem0)
    pltpu.async_copy(y2.at[sidx2.at[1]], buf1, sem1)

    def body(i, carry):
        j = 2 * i
        pltpu.make_async_copy(dsts.at[pl.ds(eb + j * _CH, _CH)], di0, sd0).wait()
        pltpu.make_async_copy(y2.at[sidx2.at[j]], buf0, sem0).wait()
        pltpu.sync_copy(buf0, agg_sh.at[di0], add=True)
        pltpu.async_copy(y2.at[sidx2.at[j + 2]], buf0, sem0)
        pltpu.async_copy(dsts.at[pl.ds(eb + (j + 2) * _CH, _CH)], di0, sd0)
        pltpu.make_async_copy(dsts.at[pl.ds(eb + (j + 1) * _CH, _CH)], di1, sd1).wait()
        pltpu.make_async_copy(y2.at[sidx2.at[j + 1]], buf1, sem1).wait()
        pltpu.sync_copy(buf1, agg_sh.at[di1], add=True)

        @pl.when(j + 3 < _NCHUNK)
        def _():
            pltpu.async_copy(y2.at[sidx2.at[j + 3]], buf1, sem1)
            pltpu.async_copy(dsts.at[pl.ds(eb + (j + 3) * _CH, _CH)], di1, sd1)
        return carry

    lax.fori_loop(0, (_NCHUNK - 1) // 2, body, 0)
    jl = _NCHUNK - 1
    pltpu.make_async_copy(dsts.at[pl.ds(eb + jl * _CH, _CH)], di0, sd0).wait()
    pltpu.make_async_copy(y2.at[sidx2.at[jl]], buf0, sem0).wait()
    pltpu.sync_copy(buf0, agg_sh.at[di0], add=True)
    plsc.subcore_barrier()
    pltpu.sync_copy(
        agg_sh.at[pl.ds(s * _ZROWS, _ZROWS)],
        agg_out.at[c, pl.ds(s * _ZROWS, _ZROWS)],
    )


def _make_agg_kernel(interpret=False):
    return functools.partial(
        pl.kernel,
        out_type=jax.ShapeDtypeStruct((2, _NPAD, _D), jnp.float32),
        mesh=_mesh,
        scratch_types=[
            pltpu.VMEM((_NCHUNK, _CH), jnp.int32),
            pltpu.VMEM((_CH,), jnp.int32),
            pltpu.VMEM((_CH,), jnp.int32),
            pltpu.VMEM((_CH, _D), jnp.float32),
            pltpu.VMEM((_CH, _D), jnp.float32),
            pltpu.VMEM_SHARED((_NPAD, _D), jnp.float32),
            pltpu.SemaphoreType.DMA,
            pltpu.SemaphoreType.DMA,
            pltpu.SemaphoreType.DMA,
            pltpu.SemaphoreType.DMA,
        ],
        interpret=interpret,
    )(_agg_body)


_agg_kernel = _make_agg_kernel()


def _scale_body(x_ref, deg_ref, y_ref):
    x3 = x_ref[...].reshape(_BB, 128, _D)
    dsum = jnp.sum(deg_ref[...], axis=2)
    cs = lax.rsqrt(jnp.maximum(dsum[:, 0], 1.0))
    y_ref[0] = (x3 * cs[0][:, :, None]).reshape(_BLK, _D)
    y_ref[1] = (x3 * cs[1][:, :, None]).reshape(_BLK, _D)


def _scale(x, deg_t):
    return pl.pallas_call(
        _scale_body,
        grid=(_GRID,),
        in_specs=[
            pl.BlockSpec((_BLK, _D), lambda i: (i, 0)),
            pl.BlockSpec((2, 2, _NS, _BB, 128), lambda i: (0, 0, 0, i, 0)),
        ],
        out_specs=pl.BlockSpec((2, _BLK, _D), lambda i: (0, i, 0)),
        out_shape=jax.ShapeDtypeStruct((2, _N, _D), jnp.float32),
    )(x, deg_t)


def _combine_body(agg_ref, deg_ref, w0_ref, w1_ref, b0_ref, b1_ref, o_ref):
    dsum = jnp.sum(deg_ref[...], axis=2)
    ci = lax.rsqrt(jnp.maximum(dsum[:, 1], 1.0))
    a = agg_ref[...]
    z0 = (a[0].reshape(_BB, 128, _D) * ci[0][:, :, None]).reshape(_BLK, _D)
    z1 = (a[1].reshape(_BB, 128, _D) * ci[1][:, :, None]).reshape(_BLK, _D)
    o_ref[...] = 0.5 * (
        jnp.dot(z0, w0_ref[...], preferred_element_type=jnp.float32)
        + jnp.dot(z1, w1_ref[...], preferred_element_type=jnp.float32)
        + (b0_ref[...] + b1_ref[...])
    )


def _combine(agg, deg_t, W0, W1, b0, b1):
    return pl.pallas_call(
        _combine_body,
        grid=(_GRID,),
        in_specs=[
            pl.BlockSpec((2, _BLK, _D), lambda i: (0, i, 0)),
            pl.BlockSpec((2, 2, _NS, _BB, 128), lambda i: (0, 0, 0, i, 0)),
            pl.BlockSpec((_D, _D), lambda i: (0, 0)),
            pl.BlockSpec((_D, _D), lambda i: (0, 0)),
            pl.BlockSpec((1, _D), lambda i: (0, 0)),
            pl.BlockSpec((1, _D), lambda i: (0, 0)),
        ],
        out_specs=pl.BlockSpec((_BLK, _D), lambda i: (i, 0)),
        out_shape=jax.ShapeDtypeStruct((_N, _D), jnp.float32),
    )(agg, deg_t, W0, W1, b0, b1)


def kernel(x, edge_index_rel0, edge_index_rel1, W0, b0, W1, b1):
    shp = (2, _NS, _NCHUNK, _CH)
    srcs4 = jnp.stack([edge_index_rel0[0], edge_index_rel1[0]]).reshape(shp)
    dsts4 = jnp.stack([edge_index_rel0[1], edge_index_rel1[1]]).reshape(shp)
    src_g4 = jnp.stack(
        [edge_index_rel0[0], edge_index_rel1[0] + _N]).reshape(shp)
    zeros_n = jnp.zeros((_NPAD,), jnp.float32)
    zbig = jnp.zeros((128, _D), jnp.float32)

    deg_t = _deg_kernel(srcs4, dsts4, zeros_n).reshape(2, 2, _NS, _NBLK, 128)
    y = _scale(x, deg_t)
    dsts_flat = jnp.concatenate([edge_index_rel0[1], edge_index_rel1[1]])
    agg = _agg_kernel(y.reshape(2 * _N, _D), src_g4, dsts_flat, zbig)
    return _combine(agg, deg_t, W0, W1, b0[None, :], b1[None, :])


# speedup vs baseline: 11.8977x; 1.1049x over previous
import functools

import jax
import jax.numpy as jnp
from jax import lax
from jax.experimental import pallas as pl
from jax.experimental.pallas import tpu as pltpu
from jax.experimental.pallas import tpu_sc as plsc

_N = 10000
_E = 160000
_D = 128

_NC = 2
_NS = 16

_EPT = _E // _NS
_CH = 80
_NG = _CH // 16
_NCHUNK = _EPT // _CH
_NPAD = 10240
_NBLK = _NPAD // 128
_ZROWS = _NPAD // _NS

_BLK = 2048
_GRID = _NPAD // _BLK
_BB = _BLK // 128

_mesh = plsc.VectorSubcoreMesh(
    core_axis_name="c", subcore_axis_name="s", num_cores=_NC, num_subcores=_NS
)


def _deg_body(srcs4, dsts4, zeros_n, deg_out, sidx2, didx2, h_src, h_dst):
    c = lax.axis_index("c")
    s = lax.axis_index("s")
    pltpu.sync_copy(srcs4.at[c, s], sidx2)
    pltpu.sync_copy(dsts4.at[c, s], didx2)
    pltpu.sync_copy(zeros_n, h_src)
    pltpu.sync_copy(zeros_n, h_dst)
    ones16 = jnp.ones((16,), jnp.float32)

    def body(j, carry):
        for g in range(_NG):
            plsc.addupdate_scatter(h_src, [sidx2[j, pl.ds(g * 16, 16)]], ones16)
            plsc.addupdate_scatter(h_dst, [didx2[j, pl.ds(g * 16, 16)]], ones16)
        return carry

    lax.fori_loop(0, _NCHUNK, body, 0)
    base = ((c * 2 + 0) * _NS + s) * _NPAD
    pltpu.sync_copy(h_src, deg_out.at[pl.ds(base, _NPAD)])
    base1 = ((c * 2 + 1) * _NS + s) * _NPAD
    pltpu.sync_copy(h_dst, deg_out.at[pl.ds(base1, _NPAD)])


def _make_deg_kernel(interpret=False):
    return functools.partial(
        pl.kernel,
        out_type=jax.ShapeDtypeStruct((2 * 2 * _NS * _NPAD,), jnp.float32),
        mesh=_mesh,
        scratch_types=[
            pltpu.VMEM((_NCHUNK, _CH), jnp.int32),
            pltpu.VMEM((_NCHUNK, _CH), jnp.int32),
            pltpu.VMEM((_NPAD,), jnp.float32),
            pltpu.VMEM((_NPAD,), jnp.float32),
        ],
        compiler_params=pltpu.CompilerParams(needs_layout_passes=False),
        interpret=interpret,
    )(_deg_body)


_deg_kernel = _make_deg_kernel()


def _agg_body(y2, src_g, dsts, zbig, agg_out,
              si0, si1, si2, di0, di1, di2, buf0, buf1, buf2, agg_sh,
              sg0, sg1, sg2, ss0, ss1, ss2,
              ssi0, ssi1, ssi2, sdi0, sdi1, sdi2):
    c = lax.axis_index("c")
    s = lax.axis_index("s")
    for k in range(_ZROWS // 128):
        pltpu.sync_copy(zbig, agg_sh.at[pl.ds(s * _ZROWS + k * 128, 128)])
    plsc.subcore_barrier()
    eb = c * _E + s * _EPT

    bufs = (buf0, buf1, buf2)
    sis = (si0, si1, si2)
    dis = (di0, di1, di2)
    sg = (sg0, sg1, sg2)
    ss = (ss0, ss1, ss2)
    ssi = (ssi0, ssi1, ssi2)
    sdi = (sdi0, sdi1, sdi2)

    def fire_si(j, b):
        pltpu.async_copy(src_g.at[pl.ds(eb + j * _CH, _CH)], sis[b], ssi[b])

    def wait_si(b):
        pltpu.make_async_copy(src_g.at[pl.ds(eb, _CH)], sis[b], ssi[b]).wait()

    def fire_di(j, b):
        pltpu.async_copy(dsts.at[pl.ds(eb + j * _CH, _CH)], dis[b], sdi[b])

    def wait_di(b):
        pltpu.make_async_copy(dsts.at[pl.ds(eb, _CH)], dis[b], sdi[b]).wait()

    def fire_gather(b):
        pltpu.async_copy(y2.at[sis[b]], bufs[b], sg[b])

    def wait_gather(b):
        pltpu.make_async_copy(y2.at[sis[b]], bufs[b], sg[b]).wait()

    def fire_scatter(b):
        pltpu.async_copy(bufs[b], agg_sh.at[dis[b]], ss[b], add=True)

    def wait_scatter(b):
        pltpu.make_async_copy(bufs[b], agg_sh.at[dis[b]], ss[b]).wait()

    fire_si(0, 0)
    fire_si(1, 1)
    fire_si(2, 2)
    fire_di(0, 0)
    fire_di(1, 1)
    wait_si(0)
    fire_gather(0)
    wait_si(1)
    fire_gather(1)

    def stage(j, b):
        b2 = (b + 2) % 3
        wait_gather(b)

        @pl.when(j + 3 < _NCHUNK)
        def _():
            fire_si(j + 3, b)
        wait_di(b)
        fire_scatter(b)

        @pl.when(j + 2 < _NCHUNK)
        def _():
            @pl.when(j >= 1)
            def _():
                wait_scatter(b2)
            fire_di(j + 2, b2)
            wait_si(b2)
            fire_gather(b2)

    def body(i, carry):
        j = 3 * i
        stage(j, 0)
        stage(j + 1, 1)
        stage(j + 2, 2)
        return carry

    lax.fori_loop(0, _NCHUNK // 3, body, 0)
    stage(_NCHUNK - 2, (_NCHUNK - 2) % 3)
    stage(_NCHUNK - 1, (_NCHUNK - 1) % 3)
    wait_scatter((_NCHUNK - 3) % 3)
    wait_scatter((_NCHUNK - 2) % 3)
    wait_scatter((_NCHUNK - 1) % 3)
    plsc.subcore_barrier()
    pltpu.sync_copy(
        agg_sh.at[pl.ds(s * _ZROWS, _ZROWS)],
        agg_out.at[c, pl.ds(s * _ZROWS, _ZROWS)],
    )


def _make_agg_kernel(interpret=False):
    return functools.partial(
        pl.kernel,
        out_type=jax.ShapeDtypeStruct((2, _NPAD, _D), jnp.float32),
        mesh=_mesh,
        scratch_types=(
            [pltpu.VMEM((_CH,), jnp.int32) for _ in range(6)]
            + [pltpu.VMEM((_CH, _D), jnp.float32) for _ in range(3)]
            + [pltpu.VMEM_SHARED((_NPAD, _D), jnp.float32)]
            + [pltpu.SemaphoreType.DMA for _ in range(12)]
        ),
        interpret=interpret,
    )(_agg_body)


_agg_kernel = _make_agg_kernel()


def _scale_body(x_ref, deg_ref, y_ref):
    x3 = x_ref[...].reshape(_BB, 128, _D)
    dsum = jnp.sum(deg_ref[...], axis=2)
    cs = lax.rsqrt(jnp.maximum(dsum[:, 0], 1.0))
    y_ref[0] = (x3 * cs[0][:, :, None]).reshape(_BLK, _D)
    y_ref[1] = (x3 * cs[1][:, :, None]).reshape(_BLK, _D)


def _scale(x, deg_t):
    return pl.pallas_call(
        _scale_body,
        grid=(_GRID,),
        in_specs=[
            pl.BlockSpec((_BLK, _D), lambda i: (i, 0)),
            pl.BlockSpec((2, 2, _NS, _BB, 128), lambda i: (0, 0, 0, i, 0)),
        ],
        out_specs=pl.BlockSpec((2, _BLK, _D), lambda i: (0, i, 0)),
        out_shape=jax.ShapeDtypeStruct((2, _N, _D), jnp.float32),
    )(x, deg_t)


def _combine_body(agg_ref, deg_ref, w0_ref, w1_ref, b0_ref, b1_ref, o_ref):
    dsum = jnp.sum(deg_ref[...], axis=2)
    ci = lax.rsqrt(jnp.maximum(dsum[:, 1], 1.0))
    a = agg_ref[...]
    z0 = (a[0].reshape(_BB, 128, _D) * ci[0][:, :, None]).reshape(_BLK, _D)
    z1 = (a[1].reshape(_BB, 128, _D) * ci[1][:, :, None]).reshape(_BLK, _D)
    o_ref[...] = 0.5 * (
        jnp.dot(z0, w0_ref[...], preferred_element_type=jnp.float32)
        + jnp.dot(z1, w1_ref[...], preferred_element_type=jnp.float32)
        + (b0_ref[...] + b1_ref[...])
    )


def _combine(agg, deg_t, W0, W1, b0, b1):
    return pl.pallas_call(
        _combine_body,
        grid=(_GRID,),
        in_specs=[
            pl.BlockSpec((2, _BLK, _D), lambda i: (0, i, 0)),
            pl.BlockSpec((2, 2, _NS, _BB, 128), lambda i: (0, 0, 0, i, 0)),
            pl.BlockSpec((_D, _D), lambda i: (0, 0)),
            pl.BlockSpec((_D, _D), lambda i: (0, 0)),
            pl.BlockSpec((1, _D), lambda i: (0, 0)),
            pl.BlockSpec((1, _D), lambda i: (0, 0)),
        ],
        out_specs=pl.BlockSpec((_BLK, _D), lambda i: (i, 0)),
        out_shape=jax.ShapeDtypeStruct((_N, _D), jnp.float32),
    )(agg, deg_t, W0, W1, b0, b1)


def kernel(x, edge_index_rel0, edge_index_rel1, W0, b0, W1, b1):
    shp = (2, _NS, _NCHUNK, _CH)
    srcs4 = jnp.stack([edge_index_rel0[0], edge_index_rel1[0]]).reshape(shp)
    dsts4 = jnp.stack([edge_index_rel0[1], edge_index_rel1[1]]).reshape(shp)
    src_g = jnp.concatenate([edge_index_rel0[0], edge_index_rel1[0] + _N])
    zeros_n = jnp.zeros((_NPAD,), jnp.float32)
    zbig = jnp.zeros((128, _D), jnp.float32)

    deg_t = _deg_kernel(srcs4, dsts4, zeros_n).reshape(2, 2, _NS, _NBLK, 128)
    y = _scale(x, deg_t)
    dsts_flat = jnp.concatenate([edge_index_rel0[1], edge_index_rel1[1]])
    agg = _agg_kernel(y.reshape(2 * _N, _D), src_g, dsts_flat, zbig)
    return _combine(agg, deg_t, W0, W1, b0[None, :], b1[None, :])
